# Initial kernel scaffold; baseline (speedup 1.0000x reference)
import jax
import jax.numpy as jnp
from jax.experimental import pallas as pl


def kernel(x, edge_index, batch, W1, b1, W2, b2, q1W, q1b, q2W, q2b, i1W, i1b, i2W, i2b):
    raise NotImplementedError("write your pallas kernel here")


# speedup vs baseline: 18.2543x; 18.2543x over previous
import functools

import jax
import jax.numpy as jnp
from jax import lax
from jax.experimental import pallas as pl
from jax.experimental.pallas import tpu as pltpu
from jax.experimental.pallas import tpu_sc as plsc

N = 10000
E = 320000
F = 128
H = 256
G = 64
A = 32

NC = 2
NS = 16
NP = 10240
RPT = NP // NS
T16 = 160
T32 = T16 // 2
EROWS = T16 * NS
EP = EROWS * 128

_mesh = plsc.VectorSubcoreMesh(core_axis_name="c", subcore_axis_name="s")


@functools.partial(
    pl.kernel,
    out_type=jax.ShapeDtypeStruct((NC * NP, 128), jnp.float32),
    mesh=_mesh,
    scratch_types=[
        pltpu.VMEM((T32, 128), jnp.int32),
        pltpu.VMEM((128, 128), jnp.float32),
        pltpu.VMEM_SHARED((NP, 128), jnp.float32),
    ],
)
def _deg_kernel(dst_hbm, zeros_hbm, ones_hbm, out_hbm, idx_v, ones_v, acc):
    cid = lax.axis_index("c")
    sid = lax.axis_index("s")
    wid = cid * NS + sid
    pltpu.sync_copy(zeros_hbm.at[pl.ds(sid * RPT, RPT)],
                    acc.at[pl.ds(sid * RPT, RPT)])
    pltpu.sync_copy(ones_hbm, ones_v)
    pltpu.sync_copy(dst_hbm.at[pl.ds(wid * T32, T32)], idx_v)
    plsc.subcore_barrier()

    def body(j, carry):
        pltpu.sync_copy(ones_v, acc.at[idx_v.at[j]], add=True)
        return carry

    lax.fori_loop(0, T32, body, 0)
    plsc.subcore_barrier()
    pltpu.sync_copy(acc.at[pl.ds(sid * RPT, RPT)],
                    out_hbm.at[pl.ds(cid * NP + sid * RPT, RPT)])


@functools.partial(
    pl.kernel,
    out_type=jax.ShapeDtypeStruct((NC * NP, 128), jnp.float32),
    mesh=_mesh,
    scratch_types=[
        pltpu.VMEM((T32, 128), jnp.int32),
        pltpu.VMEM((T32, 128), jnp.int32),
        pltpu.VMEM((128, 128), jnp.float32),
        pltpu.VMEM_SHARED((NP, 128), jnp.float32),
    ],
)
def _prop1(xs_hbm, zeros_hbm, src_hbm, dst_hbm, out_hbm, src_v, dst_v,
           rows_v, acc):
    cid = lax.axis_index("c")
    sid = lax.axis_index("s")
    wid = cid * NS + sid
    pltpu.sync_copy(zeros_hbm.at[pl.ds(sid * RPT, RPT)],
                    acc.at[pl.ds(sid * RPT, RPT)])
    pltpu.sync_copy(src_hbm.at[pl.ds(wid * T32, T32)], src_v)
    pltpu.sync_copy(dst_hbm.at[pl.ds(wid * T32, T32)], dst_v)
    plsc.subcore_barrier()

    def body(j, carry):
        pltpu.sync_copy(xs_hbm.at[src_v.at[j]], rows_v)
        pltpu.sync_copy(rows_v, acc.at[dst_v.at[j]], add=True)
        return carry

    lax.fori_loop(0, T32, body, 0)
    plsc.subcore_barrier()
    pltpu.sync_copy(acc.at[pl.ds(sid * RPT, RPT)],
                    out_hbm.at[pl.ds(cid * NP + sid * RPT, RPT)])


@functools.partial(
    pl.kernel,
    out_type=jax.ShapeDtypeStruct((NC * NP, 128), jnp.float32),
    mesh=_mesh,
    scratch_types=[
        pltpu.VMEM((32, 128), jnp.int32),
        pltpu.VMEM((32, 128), jnp.int32),
        pltpu.VMEM((128, 128), jnp.float32),
        pltpu.VMEM_SHARED((NP, 128), jnp.float32),
    ],
)
def _prop2(hs_hbm, src_hbm, dst_hbm, out_hbm, src_v, dst_v, rows_v, acc):
    cid = lax.axis_index("c")
    sid = lax.axis_index("s")
    pltpu.sync_copy(hs_hbm.at[pl.ds(cid * NP + sid * RPT, RPT)],
                    acc.at[pl.ds(sid * RPT, RPT)])
    plsc.subcore_barrier()

    def outer(blk, carry):
        pltpu.sync_copy(
            src_hbm.at[pl.ds((cid * NS + sid) * T16 + blk * 32, 32)], src_v)
        pltpu.sync_copy(
            dst_hbm.at[pl.ds(sid * T16 + blk * 32, 32)], dst_v)

        def body(j, c2):
            pltpu.sync_copy(hs_hbm.at[src_v.at[j]], rows_v)
            pltpu.sync_copy(rows_v, acc.at[dst_v.at[j]], add=True)
            return c2

        return lax.fori_loop(0, 32, body, carry)

    lax.fori_loop(0, T16 // 32, outer, 0)
    plsc.subcore_barrier()
    pltpu.sync_copy(acc.at[pl.ds(sid * RPT, RPT)],
                    out_hbm.at[pl.ds(cid * NP + sid * RPT, RPT)])


_NB = 8
_BR = NP // _NB


def _prep1_body(x_ref, dega_ref, degb_ref, out_ref):
    deg = dega_ref[:, :1] + degb_ref[:, :1] + 1.0
    dis = lax.rsqrt(deg)
    out_ref[...] = x_ref[...] * dis


def _prep1(x_pad, degs):
    return pl.pallas_call(
        _prep1_body,
        grid=(_NB,),
        in_specs=[
            pl.BlockSpec((_BR, F), lambda i: (i, 0)),
            pl.BlockSpec((_BR, 128), lambda i: (i, 0)),
            pl.BlockSpec((_BR, 128), lambda i: (i + _NB, 0)),
        ],
        out_specs=pl.BlockSpec((_BR, F), lambda i: (i, 0)),
        out_shape=jax.ShapeDtypeStruct((NP, F), jnp.float32),
    )(x_pad, degs, degs)


def _mid_body(a_ref, b_ref, x_ref, dega_ref, degb_ref, w_ref, bias_ref,
              out_ref):
    deg = dega_ref[:, :1] + degb_ref[:, :1] + 1.0
    dis = lax.rsqrt(deg)
    y = (a_ref[...] + b_ref[...] + x_ref[...] * dis) * dis
    h = jnp.dot(y, w_ref[...], preferred_element_type=jnp.float32)
    h = jnp.maximum(h + bias_ref[...], 0.0) * dis
    out_ref[...] = jnp.stack([h[:, :128], h[:, 128:]], axis=0)


def _mid(acc1, x_pad, degs, W1, b1):
    return pl.pallas_call(
        _mid_body,
        grid=(_NB,),
        in_specs=[
            pl.BlockSpec((_BR, 128), lambda i: (i, 0)),
            pl.BlockSpec((_BR, 128), lambda i: (i + _NB, 0)),
            pl.BlockSpec((_BR, F), lambda i: (i, 0)),
            pl.BlockSpec((_BR, 128), lambda i: (i, 0)),
            pl.BlockSpec((_BR, 128), lambda i: (i + _NB, 0)),
            pl.BlockSpec((F, H), lambda i: (0, 0)),
            pl.BlockSpec((1, H), lambda i: (0, 0)),
        ],
        out_specs=pl.BlockSpec((2, _BR, 128), lambda i: (0, i, 0)),
        out_shape=jax.ShapeDtypeStruct((2, NP, 128), jnp.float32),
    )(acc1, acc1, x_pad, degs, degs, W1, b1)


def _pool_body(a_ref, b_ref, dega_ref, degb_ref, bt_ref, psum_ref, cnt_ref):
    i = pl.program_id(0)

    @pl.when(i == 0)
    def _():
        psum_ref[...] = jnp.zeros_like(psum_ref)
        cnt_ref[...] = jnp.zeros_like(cnt_ref)

    deg = dega_ref[:, :1] + degb_ref[:, :1] + 1.0
    dis = lax.rsqrt(deg)
    y = jnp.concatenate([a_ref[...], b_ref[...]], axis=1) * dis
    bt = bt_ref[...]
    gi = lax.broadcasted_iota(jnp.int32, (G, _BR), 0)
    m = (bt == gi).astype(jnp.float32)
    psum_ref[...] += jnp.dot(m, y, preferred_element_type=jnp.float32)
    cnt_ref[...] += jnp.broadcast_to(jnp.sum(m, axis=1, keepdims=True),
                                     (G, 128))


def _pool(acc2, degs, batch2d):
    return pl.pallas_call(
        _pool_body,
        grid=(_NB,),
        in_specs=[
            pl.BlockSpec((_BR, 128), lambda i: (i, 0)),
            pl.BlockSpec((_BR, 128), lambda i: (i + _NB, 0)),
            pl.BlockSpec((_BR, 128), lambda i: (i, 0)),
            pl.BlockSpec((_BR, 128), lambda i: (i + _NB, 0)),
            pl.BlockSpec((1, _BR), lambda i: (0, i)),
        ],
        out_specs=[
            pl.BlockSpec((G, H), lambda i: (0, 0)),
            pl.BlockSpec((G, 128), lambda i: (0, 0)),
        ],
        out_shape=[
            jax.ShapeDtypeStruct((G, H), jnp.float32),
            jax.ShapeDtypeStruct((G, 128), jnp.float32),
        ],
    )(acc2, acc2, degs, degs, batch2d)


def _head_body(ps_ref, cnt_ref, w2_ref, b2_ref, q1w_ref, q1b_ref, q2w_ref,
               q2b_ref, i1w_ref, i1b_ref, i2w_ref, i2b_ref,
               q_ref, lsm_ref, i_ref):
    cnt = jnp.maximum(cnt_ref[:, :1], 1.0)
    pooled = jnp.dot(ps_ref[...] / cnt, w2_ref[...],
                     preferred_element_type=jnp.float32) + b2_ref[...]
    qh = jnp.maximum(
        jnp.dot(pooled, q1w_ref[...], preferred_element_type=jnp.float32)
        + q1b_ref[...], 0.0)
    q = jnp.dot(qh, q2w_ref[...],
                preferred_element_type=jnp.float32) + q2b_ref[...]
    ih = jnp.maximum(
        jnp.dot(pooled, i1w_ref[...], preferred_element_type=jnp.float32)
        + i1b_ref[...], 0.0)
    ii = jnp.dot(ih, i2w_ref[...],
                 preferred_element_type=jnp.float32) + i2b_ref[...]
    mx = jnp.max(ii, axis=1, keepdims=True)
    lse = jnp.log(jnp.sum(jnp.exp(ii - mx), axis=1, keepdims=True)) + mx
    q_ref[...] = q
    lsm_ref[...] = ii - lse
    i_ref[...] = ii


def _head(psum, cnt, W2, b2, q1W, q1b, q2W, q2b, i1W, i1b, i2W, i2b):
    return pl.pallas_call(
        _head_body,
        out_shape=[
            jax.ShapeDtypeStruct((G, A), jnp.float32),
            jax.ShapeDtypeStruct((G, A), jnp.float32),
            jax.ShapeDtypeStruct((G, A), jnp.float32),
        ],
    )(psum, cnt, W2, b2, q1W, q1b, q2W, q2b, i1W, i1b, i2W, i2b)


def kernel(x, edge_index, batch, W1, b1, W2, b2, q1W, q1b, q2W, q2b,
           i1W, i1b, i2W, i2b):
    f32 = jnp.float32
    src = edge_index[0]
    dst = edge_index[1]
    pad = EP - E
    padi = jnp.arange(pad, dtype=jnp.int32)
    pad_rows = N + (padi % (NP - N))
    src2d = jnp.concatenate([src, pad_rows]).reshape(EROWS, 128)
    dst2d = jnp.concatenate([dst, pad_rows]).reshape(EROWS, 128)
    src_both = jnp.concatenate([src2d, src2d + NP], axis=0)

    zeros = jnp.zeros((NP, 128), f32)
    ones = jnp.ones((128, 128), f32)
    degs = _deg_kernel(dst2d, zeros, ones)

    x_pad = jnp.pad(x, ((0, NP - N), (0, 0)))
    xs1 = _prep1(x_pad, degs)
    acc1 = _prop1(xs1, zeros, src2d, dst2d)
    hs = _mid(acc1, x_pad, degs, W1, b1.reshape(1, H)).reshape(NC * NP, 128)
    acc2 = _prop2(hs, src_both, dst2d)

    batch2d = jnp.pad(batch, (0, NP - N), constant_values=G).reshape(1, NP)
    psum, cnt = _pool(acc2, degs, batch2d)
    q, lsm, ii = _head(psum, cnt, W2, b2.reshape(1, H),
                       q1W, q1b.reshape(1, 64), q2W, q2b.reshape(1, A),
                       i1W, i1b.reshape(1, 64), i2W, i2b.reshape(1, A))
    return (q, lsm, ii)


# speedup vs baseline: 26.1042x; 1.4300x over previous
import functools

import jax
import jax.numpy as jnp
from jax import lax
from jax.experimental import pallas as pl
from jax.experimental.pallas import tpu as pltpu
from jax.experimental.pallas import tpu_sc as plsc

N = 10000
E = 320000
F = 128
H = 256
G = 64
A = 32

NC = 2
NS = 16
NP = 10240
RPT = NP // NS
T16 = 160
T32 = T16 // 2
EROWS = T16 * NS
EP = EROWS * 128

_mesh = plsc.VectorSubcoreMesh(core_axis_name="c", subcore_axis_name="s")


@functools.partial(
    pl.kernel,
    out_type=jax.ShapeDtypeStruct((NC * NP, 128), jnp.float32),
    mesh=_mesh,
    scratch_types=[
        pltpu.VMEM((T32, 128), jnp.int32),
        pltpu.VMEM((128, 128), jnp.float32),
        pltpu.VMEM_SHARED((NP, 128), jnp.float32),
        pltpu.SemaphoreType.DMA,
    ],
)
def _deg_kernel(dst_hbm, zeros_hbm, ones_hbm, out_hbm, idx_v, ones_v, acc,
                dsem):
    cid = lax.axis_index("c")
    sid = lax.axis_index("s")
    wid = cid * NS + sid
    pltpu.sync_copy(zeros_hbm.at[pl.ds(sid * RPT, RPT)],
                    acc.at[pl.ds(sid * RPT, RPT)])
    pltpu.sync_copy(ones_hbm, ones_v)
    pltpu.sync_copy(dst_hbm.at[pl.ds(wid * T32, T32)], idx_v)
    plsc.subcore_barrier()

    def body(q, carry):
        base = q * 4
        descs = [
            pltpu.async_copy(ones_v, acc.at[idx_v.at[base + k]], dsem,
                             add=True)
            for k in range(4)
        ]
        for d in descs:
            d.wait()
        return carry

    lax.fori_loop(0, T32 // 4, body, 0)
    plsc.subcore_barrier()
    pltpu.sync_copy(acc.at[pl.ds(sid * RPT, RPT)],
                    out_hbm.at[pl.ds(cid * NP + sid * RPT, RPT)])


def _edge_pipeline(tab_hbm, src_hbm, dst_hbm, acc, src_v, dst_v, rows_a,
                   rows_b, gsa, gsb, ssa, ssb, src_base, dst_base, nblk,
                   blk_rows):
    npair = blk_rows // 2

    def outer(blk, carry):
        pltpu.sync_copy(
            src_hbm.at[pl.ds(src_base + blk * blk_rows, blk_rows)], src_v)
        pltpu.sync_copy(
            dst_hbm.at[pl.ds(dst_base + blk * blk_rows, blk_rows)], dst_v)
        pltpu.async_copy(tab_hbm.at[src_v.at[0]], rows_a, gsa)

        def pair(t, c2):
            j0 = 2 * t
            j1 = j0 + 1
            pltpu.async_copy(tab_hbm.at[src_v.at[j1]], rows_b, gsb)
            pltpu.make_async_copy(tab_hbm.at[src_v.at[j0]], rows_a,
                                  gsa).wait()
            sa = pltpu.async_copy(rows_a, acc.at[dst_v.at[j0]], ssa,
                                  add=True)
            sa.wait()

            @pl.when(t < npair - 1)
            def _():
                pltpu.async_copy(tab_hbm.at[src_v.at[j0 + 2]], rows_a, gsa)

            pltpu.make_async_copy(tab_hbm.at[src_v.at[j1]], rows_b,
                                  gsb).wait()
            sb = pltpu.async_copy(rows_b, acc.at[dst_v.at[j1]], ssb,
                                  add=True)
            sb.wait()
            return c2

        return lax.fori_loop(0, npair, pair, carry)

    lax.fori_loop(0, nblk, outer, 0)


@functools.partial(
    pl.kernel,
    out_type=jax.ShapeDtypeStruct((NC * NP, 128), jnp.float32),
    mesh=_mesh,
    scratch_types=[
        pltpu.VMEM((40, 128), jnp.int32),
        pltpu.VMEM((40, 128), jnp.int32),
        pltpu.VMEM((128, 128), jnp.float32),
        pltpu.VMEM((128, 128), jnp.float32),
        pltpu.VMEM_SHARED((NP, 128), jnp.float32),
        pltpu.SemaphoreType.DMA,
        pltpu.SemaphoreType.DMA,
        pltpu.SemaphoreType.DMA,
        pltpu.SemaphoreType.DMA,
    ],
)
def _prop1(xs_hbm, zeros_hbm, src_hbm, dst_hbm, out_hbm, src_v, dst_v,
           rows_a, rows_b, acc, gsa, gsb, ssa, ssb):
    cid = lax.axis_index("c")
    sid = lax.axis_index("s")
    wid = cid * NS + sid
    pltpu.sync_copy(zeros_hbm.at[pl.ds(sid * RPT, RPT)],
                    acc.at[pl.ds(sid * RPT, RPT)])
    plsc.subcore_barrier()
    _edge_pipeline(xs_hbm, src_hbm, dst_hbm, acc, src_v, dst_v, rows_a,
                   rows_b, gsa, gsb, ssa, ssb,
                   src_base=wid * T32, dst_base=wid * T32, nblk=2,
                   blk_rows=40)
    plsc.subcore_barrier()
    pltpu.sync_copy(acc.at[pl.ds(sid * RPT, RPT)],
                    out_hbm.at[pl.ds(cid * NP + sid * RPT, RPT)])


@functools.partial(
    pl.kernel,
    out_type=jax.ShapeDtypeStruct((NC * NP, 128), jnp.float32),
    mesh=_mesh,
    scratch_types=[
        pltpu.VMEM((32, 128), jnp.int32),
        pltpu.VMEM((32, 128), jnp.int32),
        pltpu.VMEM((128, 128), jnp.float32),
        pltpu.VMEM((128, 128), jnp.float32),
        pltpu.VMEM_SHARED((NP, 128), jnp.float32),
        pltpu.SemaphoreType.DMA,
        pltpu.SemaphoreType.DMA,
        pltpu.SemaphoreType.DMA,
        pltpu.SemaphoreType.DMA,
    ],
)
def _prop2(hs_hbm, src_hbm, dst_hbm, out_hbm, src_v, dst_v, rows_a, rows_b,
           acc, gsa, gsb, ssa, ssb):
    cid = lax.axis_index("c")
    sid = lax.axis_index("s")
    pltpu.sync_copy(hs_hbm.at[pl.ds(cid * NP + sid * RPT, RPT)],
                    acc.at[pl.ds(sid * RPT, RPT)])
    plsc.subcore_barrier()
    _edge_pipeline(hs_hbm, src_hbm, dst_hbm, acc, src_v, dst_v, rows_a,
                   rows_b, gsa, gsb, ssa, ssb,
                   src_base=(cid * NS + sid) * T16, dst_base=sid * T16,
                   nblk=T16 // 32, blk_rows=32)
    plsc.subcore_barrier()
    pltpu.sync_copy(acc.at[pl.ds(sid * RPT, RPT)],
                    out_hbm.at[pl.ds(cid * NP + sid * RPT, RPT)])


_NB = 8
_BR = NP // _NB


def _prep1_body(x_ref, dega_ref, degb_ref, xs_ref, dis_ref):
    deg = dega_ref[:, :1] + degb_ref[:, :1] + 1.0
    dis = lax.rsqrt(deg)
    xs_ref[...] = x_ref[...] * dis
    dis_ref[...] = jnp.broadcast_to(dis, (_BR, 8))


def _prep1(x_pad, degs):
    return pl.pallas_call(
        _prep1_body,
        grid=(_NB,),
        in_specs=[
            pl.BlockSpec((_BR, F), lambda i: (i, 0)),
            pl.BlockSpec((_BR, 128), lambda i: (i, 0)),
            pl.BlockSpec((_BR, 128), lambda i: (i + _NB, 0)),
        ],
        out_specs=[
            pl.BlockSpec((_BR, F), lambda i: (i, 0)),
            pl.BlockSpec((_BR, 8), lambda i: (i, 0)),
        ],
        out_shape=[
            jax.ShapeDtypeStruct((NP, F), jnp.float32),
            jax.ShapeDtypeStruct((NP, 8), jnp.float32),
        ],
    )(x_pad, degs, degs)


def _mid_body(a_ref, b_ref, xs_ref, dis_ref, w_ref, bias_ref, out_ref):
    dis = dis_ref[:, :1]
    y = (a_ref[...] + b_ref[...] + xs_ref[...]) * dis
    h = jnp.dot(y, w_ref[...], preferred_element_type=jnp.float32)
    h = jnp.maximum(h + bias_ref[...], 0.0) * dis
    out_ref[...] = jnp.stack([h[:, :128], h[:, 128:]], axis=0)


def _mid(acc1, xs1, dis8, W1, b1):
    return pl.pallas_call(
        _mid_body,
        grid=(_NB,),
        in_specs=[
            pl.BlockSpec((_BR, 128), lambda i: (i, 0)),
            pl.BlockSpec((_BR, 128), lambda i: (i + _NB, 0)),
            pl.BlockSpec((_BR, F), lambda i: (i, 0)),
            pl.BlockSpec((_BR, 8), lambda i: (i, 0)),
            pl.BlockSpec((F, H), lambda i: (0, 0)),
            pl.BlockSpec((1, H), lambda i: (0, 0)),
        ],
        out_specs=pl.BlockSpec((2, _BR, 128), lambda i: (0, i, 0)),
        out_shape=jax.ShapeDtypeStruct((2, NP, 128), jnp.float32),
    )(acc1, acc1, xs1, dis8, W1, b1)


def _pool_body(a_ref, b_ref, dis_ref, bt_ref, psum_ref, cnt_ref):
    i = pl.program_id(0)

    @pl.when(i == 0)
    def _():
        psum_ref[...] = jnp.zeros_like(psum_ref)
        cnt_ref[...] = jnp.zeros_like(cnt_ref)

    dis = dis_ref[:, :1]
    y = jnp.concatenate([a_ref[...], b_ref[...]], axis=1) * dis
    bt = bt_ref[...]
    gi = lax.broadcasted_iota(jnp.int32, (G, _BR), 0)
    m = (bt == gi).astype(jnp.float32)
    psum_ref[...] += jnp.dot(m, y, preferred_element_type=jnp.float32)
    cnt_ref[...] += jnp.broadcast_to(jnp.sum(m, axis=1, keepdims=True),
                                     (G, 128))


def _pool(acc2, dis8, batch2d):
    return pl.pallas_call(
        _pool_body,
        grid=(_NB,),
        in_specs=[
            pl.BlockSpec((_BR, 128), lambda i: (i, 0)),
            pl.BlockSpec((_BR, 128), lambda i: (i + _NB, 0)),
            pl.BlockSpec((_BR, 8), lambda i: (i, 0)),
            pl.BlockSpec((1, _BR), lambda i: (0, i)),
        ],
        out_specs=[
            pl.BlockSpec((G, H), lambda i: (0, 0)),
            pl.BlockSpec((G, 128), lambda i: (0, 0)),
        ],
        out_shape=[
            jax.ShapeDtypeStruct((G, H), jnp.float32),
            jax.ShapeDtypeStruct((G, 128), jnp.float32),
        ],
    )(acc2, acc2, dis8, batch2d)


def _head_body(ps_ref, cnt_ref, w2_ref, b2_ref, q1w_ref, q1b_ref, q2w_ref,
               q2b_ref, i1w_ref, i1b_ref, i2w_ref, i2b_ref,
               q_ref, lsm_ref, i_ref):
    cnt = jnp.maximum(cnt_ref[:, :1], 1.0)
    pooled = jnp.dot(ps_ref[...] / cnt, w2_ref[...],
                     preferred_element_type=jnp.float32) + b2_ref[...]
    qh = jnp.maximum(
        jnp.dot(pooled, q1w_ref[...], preferred_element_type=jnp.float32)
        + q1b_ref[...], 0.0)
    q = jnp.dot(qh, q2w_ref[...],
                preferred_element_type=jnp.float32) + q2b_ref[...]
    ih = jnp.maximum(
        jnp.dot(pooled, i1w_ref[...], preferred_element_type=jnp.float32)
        + i1b_ref[...], 0.0)
    ii = jnp.dot(ih, i2w_ref[...],
                 preferred_element_type=jnp.float32) + i2b_ref[...]
    mx = jnp.max(ii, axis=1, keepdims=True)
    lse = jnp.log(jnp.sum(jnp.exp(ii - mx), axis=1, keepdims=True)) + mx
    q_ref[...] = q
    lsm_ref[...] = ii - lse
    i_ref[...] = ii


def _head(psum, cnt, W2, b2, q1W, q1b, q2W, q2b, i1W, i1b, i2W, i2b):
    return pl.pallas_call(
        _head_body,
        out_shape=[
            jax.ShapeDtypeStruct((G, A), jnp.float32),
            jax.ShapeDtypeStruct((G, A), jnp.float32),
            jax.ShapeDtypeStruct((G, A), jnp.float32),
        ],
    )(psum, cnt, W2, b2, q1W, q1b, q2W, q2b, i1W, i1b, i2W, i2b)


def kernel(x, edge_index, batch, W1, b1, W2, b2, q1W, q1b, q2W, q2b,
           i1W, i1b, i2W, i2b):
    f32 = jnp.float32
    src = edge_index[0]
    dst = edge_index[1]
    pad = EP - E
    padi = jnp.arange(pad, dtype=jnp.int32)
    pad_rows = N + (padi % (NP - N))
    src2d = jnp.concatenate([src, pad_rows]).reshape(EROWS, 128)
    dst2d = jnp.concatenate([dst, pad_rows]).reshape(EROWS, 128)
    src_both = jnp.concatenate([src2d, src2d + NP], axis=0)

    zeros = jnp.zeros((NP, 128), f32)
    ones = jnp.ones((128, 128), f32)
    degs = _deg_kernel(dst2d, zeros, ones)

    x_pad = jnp.pad(x, ((0, NP - N), (0, 0)))
    xs1, dis8 = _prep1(x_pad, degs)
    acc1 = _prop1(xs1, zeros, src2d, dst2d)
    hs = _mid(acc1, xs1, dis8, W1, b1.reshape(1, H)).reshape(NC * NP, 128)
    acc2 = _prop2(hs, src_both, dst2d)

    batch2d = jnp.pad(batch, (0, NP - N), constant_values=G).reshape(1, NP)
    psum, cnt = _pool(acc2, dis8, batch2d)
    q, lsm, ii = _head(psum, cnt, W2, b2.reshape(1, H),
                       q1W, q1b.reshape(1, 64), q2W, q2b.reshape(1, A),
                       i1W, i1b.reshape(1, 64), i2W, i2b.reshape(1, A))
    return (q, lsm, ii)


# speedup vs baseline: 50.0590x; 1.9177x over previous
import functools

import jax
import jax.numpy as jnp
from jax import lax
from jax.experimental import pallas as pl
from jax.experimental.pallas import tpu as pltpu
from jax.experimental.pallas import tpu_sc as plsc

N = 10000
E = 320000
F = 128
H = 256
G = 64
A = 32

NC = 2
NS = 16
NP = 10240
RPT = NP // NS
T16 = 160
T32 = T16 // 2
EROWS = T16 * NS
EP = EROWS * 128

_mesh = plsc.VectorSubcoreMesh(core_axis_name="c", subcore_axis_name="s")


_DRT = 8 * NP // NS


@functools.partial(
    pl.kernel,
    out_type=jax.ShapeDtypeStruct((NC * 8 * NP, ), jnp.float32),
    mesh=_mesh,
    scratch_types=[
        pltpu.VMEM((T32, 128), jnp.int32),
        pltpu.VMEM((128, ), jnp.float32),
        pltpu.VMEM_SHARED((8 * NP, ), jnp.float32),
        pltpu.SemaphoreType.DMA,
    ],
)
def _deg_kernel(dst8_hbm, zeros_hbm, ones_hbm, out_hbm, idx_v, ones_v, acc,
                dsem):
    cid = lax.axis_index("c")
    sid = lax.axis_index("s")
    wid = cid * NS + sid
    pltpu.sync_copy(zeros_hbm.at[pl.ds(sid * _DRT, _DRT)],
                    acc.at[pl.ds(sid * _DRT, _DRT)])
    pltpu.sync_copy(ones_hbm, ones_v)
    pltpu.sync_copy(dst8_hbm.at[pl.ds(wid * T32, T32)], idx_v)
    plsc.subcore_barrier()

    def body(q, carry):
        base = q * 4
        descs = [
            pltpu.async_copy(ones_v, acc.at[idx_v.at[base + k]], dsem,
                             add=True)
            for k in range(4)
        ]
        for d in descs:
            d.wait()
        return carry

    lax.fori_loop(0, T32 // 4, body, 0)
    plsc.subcore_barrier()
    pltpu.sync_copy(acc.at[pl.ds(sid * _DRT, _DRT)],
                    out_hbm.at[pl.ds(cid * 8 * NP + sid * _DRT, _DRT)])


def _edge_pipeline(tab_hbm, src_hbm, dst_hbm, acc, src_v, dst_v, rows_a,
                   rows_b, gsa, gsb, ssa, ssb, src_base, dst_base, nblk,
                   blk_rows):
    npair = blk_rows // 2

    def outer(blk, carry):
        pltpu.sync_copy(
            src_hbm.at[pl.ds(src_base + blk * blk_rows, blk_rows)], src_v)
        pltpu.sync_copy(
            dst_hbm.at[pl.ds(dst_base + blk * blk_rows, blk_rows)], dst_v)
        pltpu.async_copy(tab_hbm.at[src_v.at[0]], rows_a, gsa)

        def pair(t, c2):
            j0 = 2 * t
            j1 = j0 + 1
            pltpu.async_copy(tab_hbm.at[src_v.at[j1]], rows_b, gsb)
            pltpu.make_async_copy(tab_hbm.at[src_v.at[j0]], rows_a,
                                  gsa).wait()
            sa = pltpu.async_copy(rows_a, acc.at[dst_v.at[j0]], ssa,
                                  add=True)
            sa.wait()

            @pl.when(t < npair - 1)
            def _():
                pltpu.async_copy(tab_hbm.at[src_v.at[j0 + 2]], rows_a, gsa)

            pltpu.make_async_copy(tab_hbm.at[src_v.at[j1]], rows_b,
                                  gsb).wait()
            sb = pltpu.async_copy(rows_b, acc.at[dst_v.at[j1]], ssb,
                                  add=True)
            sb.wait()
            return c2

        return lax.fori_loop(0, npair, pair, carry)

    lax.fori_loop(0, nblk, outer, 0)


@functools.partial(
    pl.kernel,
    out_type=jax.ShapeDtypeStruct((NC * NP, 128), jnp.float32),
    mesh=_mesh,
    scratch_types=[
        pltpu.VMEM((40, 128), jnp.int32),
        pltpu.VMEM((40, 128), jnp.int32),
        pltpu.VMEM((128, 128), jnp.float32),
        pltpu.VMEM((128, 128), jnp.float32),
        pltpu.VMEM_SHARED((NP, 128), jnp.float32),
        pltpu.SemaphoreType.DMA,
        pltpu.SemaphoreType.DMA,
        pltpu.SemaphoreType.DMA,
        pltpu.SemaphoreType.DMA,
    ],
)
def _prop1(xs_hbm, src_hbm, dst_hbm, out_hbm, src_v, dst_v,
           rows_a, rows_b, acc, gsa, gsb, ssa, ssb):
    cid = lax.axis_index("c")
    sid = lax.axis_index("s")
    wid = cid * NS + sid
    pltpu.sync_copy(xs_hbm.at[pl.ds(sid * RPT, RPT)],
                    acc.at[pl.ds(sid * RPT, RPT)])
    plsc.subcore_barrier()
    _edge_pipeline(xs_hbm, src_hbm, dst_hbm, acc, src_v, dst_v, rows_a,
                   rows_b, gsa, gsb, ssa, ssb,
                   src_base=wid * T32, dst_base=wid * T32, nblk=2,
                   blk_rows=40)
    plsc.subcore_barrier()
    pltpu.sync_copy(acc.at[pl.ds(sid * RPT, RPT)],
                    out_hbm.at[pl.ds(cid * NP + sid * RPT, RPT)])


_URT = 64 * NP // NS
_UZT = 65 * NP // NS
_NPS = NP // NS


@functools.partial(
    pl.kernel,
    out_type=jax.ShapeDtypeStruct((NC * 64 * NP, ), jnp.float32),
    mesh=_mesh,
    scratch_types=[
        pltpu.VMEM((T32, 128), jnp.int32),
        pltpu.VMEM((T32, 128), jnp.int32),
        pltpu.VMEM((128, ), jnp.int32),
        pltpu.VMEM((128, ), jnp.float32),
        pltpu.VMEM((128, ), jnp.int32),
        pltpu.VMEM((128, ), jnp.int32),
        pltpu.VMEM((128, ), jnp.float32),
        pltpu.VMEM((128, ), jnp.int32),
        pltpu.VMEM_SHARED((65 * NP, ), jnp.float32),
        pltpu.VMEM_SHARED((NP, ), jnp.int32),
        pltpu.VMEM_SHARED((NP, ), jnp.float32),
        pltpu.SemaphoreType.DMA,
        pltpu.SemaphoreType.DMA,
        pltpu.SemaphoreType.DMA,
        pltpu.SemaphoreType.DMA,
        pltpu.SemaphoreType.DMA,
        pltpu.SemaphoreType.DMA,
    ],
)
def _upass(key_hbm, dis_hbm, zeros_hbm, src_hbm, dst_hbm, out_hbm,
           dst_v, src_v, krow_a, vrow_a, flat_a, krow_b, vrow_b, flat_b,
           acc, key_sp, dis_sp, gka, gva, gkb, gvb, sca, scb):
    cid = lax.axis_index("c")
    sid = lax.axis_index("s")
    wid = cid * NS + sid
    pltpu.sync_copy(zeros_hbm.at[pl.ds(sid * _UZT, _UZT)],
                    acc.at[pl.ds(sid * _UZT, _UZT)])
    pltpu.sync_copy(key_hbm.at[pl.ds(sid * _NPS, _NPS)],
                    key_sp.at[pl.ds(sid * _NPS, _NPS)])
    pltpu.sync_copy(dis_hbm.at[pl.ds(sid * _NPS, _NPS)],
                    dis_sp.at[pl.ds(sid * _NPS, _NPS)])
    pltpu.sync_copy(dst_hbm.at[pl.ds(wid * T32, T32)], dst_v)
    pltpu.sync_copy(src_hbm.at[pl.ds(wid * T32, T32)], src_v)
    plsc.subcore_barrier()

    def _half(j, krow, vrow, flat_v, gk, gv, sc):
        pltpu.make_async_copy(key_sp.at[dst_v.at[j]], krow, gk).wait()
        pltpu.make_async_copy(dis_sp.at[dst_v.at[j]], vrow, gv).wait()
        for k in range(8):
            sl = pl.ds(k * 16, 16)
            flat_v[sl] = krow[sl] + src_v[j, sl]
        pltpu.async_copy(vrow, acc.at[flat_v], sc, add=True).wait()

    pltpu.async_copy(key_sp.at[dst_v.at[0]], krow_a, gka)
    pltpu.async_copy(dis_sp.at[dst_v.at[0]], vrow_a, gva)

    def pair(t, carry):
        j0 = 2 * t
        j1 = j0 + 1
        pltpu.async_copy(key_sp.at[dst_v.at[j1]], krow_b, gkb)
        pltpu.async_copy(dis_sp.at[dst_v.at[j1]], vrow_b, gvb)
        _half(j0, krow_a, vrow_a, flat_a, gka, gva, sca)

        @pl.when(t < T32 // 2 - 1)
        def _():
            pltpu.async_copy(key_sp.at[dst_v.at[j0 + 2]], krow_a, gka)
            pltpu.async_copy(dis_sp.at[dst_v.at[j0 + 2]], vrow_a, gva)

        _half(j1, krow_b, vrow_b, flat_b, gkb, gvb, scb)
        return carry

    lax.fori_loop(0, T32 // 2, pair, 0)
    plsc.subcore_barrier()
    pltpu.sync_copy(acc.at[pl.ds(sid * _URT, _URT)],
                    out_hbm.at[pl.ds(cid * 64 * NP + sid * _URT, _URT)])


_NB = 8
_BR = NP // _NB


def _prep1_body(x_ref, dega_ref, degb_ref, xs_ref, dis_ref):
    deg = dega_ref[0][:, :1] + degb_ref[0][:, :1] + 1.0
    dis = lax.rsqrt(deg)
    xs_ref[...] = x_ref[...] * dis
    dis_ref[...] = jnp.broadcast_to(dis, (_BR, 8))


def _prep1(x_pad, degs3):
    return pl.pallas_call(
        _prep1_body,
        grid=(_NB,),
        in_specs=[
            pl.BlockSpec((_BR, F), lambda i: (i, 0)),
            pl.BlockSpec((1, _BR, 8), lambda i: (0, i, 0)),
            pl.BlockSpec((1, _BR, 8), lambda i: (1, i, 0)),
        ],
        out_specs=[
            pl.BlockSpec((_BR, F), lambda i: (i, 0)),
            pl.BlockSpec((_BR, 8), lambda i: (i, 0)),
        ],
        out_shape=[
            jax.ShapeDtypeStruct((NP, F), jnp.float32),
            jax.ShapeDtypeStruct((NP, 8), jnp.float32),
        ],
    )(x_pad, degs3, degs3)


def _mid_body(a_ref, b_ref, xs_ref, dis_ref, w_ref, bias_ref, out_ref):
    dis = dis_ref[:, :1]
    y = (a_ref[...] + b_ref[...] - xs_ref[...]) * dis
    h = jnp.dot(y, w_ref[...], preferred_element_type=jnp.float32)
    out_ref[...] = jnp.maximum(h + bias_ref[...], 0.0) * dis


def _mid(acc1, xs1, dis8, W1, b1):
    return pl.pallas_call(
        _mid_body,
        grid=(_NB,),
        in_specs=[
            pl.BlockSpec((_BR, 128), lambda i: (i, 0)),
            pl.BlockSpec((_BR, 128), lambda i: (i + _NB, 0)),
            pl.BlockSpec((_BR, F), lambda i: (i, 0)),
            pl.BlockSpec((_BR, 8), lambda i: (i, 0)),
            pl.BlockSpec((F, H), lambda i: (0, 0)),
            pl.BlockSpec((1, H), lambda i: (0, 0)),
        ],
        out_specs=pl.BlockSpec((_BR, H), lambda i: (i, 0)),
        out_shape=jax.ShapeDtypeStruct((NP, H), jnp.float32),
    )(acc1, acc1, xs1, dis8, W1, b1)


def _pool_body(hs_ref, u0_ref, u1_ref, dis_ref, bt_ref, psum_ref, cnt_ref):
    i = pl.program_id(0)

    @pl.when(i == 0)
    def _():
        psum_ref[...] = jnp.zeros_like(psum_ref)
        cnt_ref[...] = jnp.zeros_like(cnt_ref)

    dis = dis_ref[:, :1]
    y = hs_ref[...]
    bt = bt_ref[...]
    gi = lax.broadcasted_iota(jnp.int32, (G, _BR), 0)
    m = (bt == gi).astype(jnp.float32)
    u = u0_ref[0] + u1_ref[0]
    psum_ref[...] += (
        jnp.dot(u, y, preferred_element_type=jnp.float32)
        + jnp.dot(m, y * dis, preferred_element_type=jnp.float32))
    cnt_ref[...] += jnp.broadcast_to(jnp.sum(m, axis=1, keepdims=True),
                                     (G, 128))


def _pool(hs, u3, dis8, batch2d):
    return pl.pallas_call(
        _pool_body,
        grid=(_NB,),
        in_specs=[
            pl.BlockSpec((_BR, H), lambda i: (i, 0)),
            pl.BlockSpec((1, G, _BR), lambda i: (0, 0, i)),
            pl.BlockSpec((1, G, _BR), lambda i: (1, 0, i)),
            pl.BlockSpec((_BR, 8), lambda i: (i, 0)),
            pl.BlockSpec((1, _BR), lambda i: (0, i)),
        ],
        out_specs=[
            pl.BlockSpec((G, H), lambda i: (0, 0)),
            pl.BlockSpec((G, 128), lambda i: (0, 0)),
        ],
        out_shape=[
            jax.ShapeDtypeStruct((G, H), jnp.float32),
            jax.ShapeDtypeStruct((G, 128), jnp.float32),
        ],
    )(hs, u3, u3, dis8, batch2d)


def _head_body(ps_ref, cnt_ref, w2_ref, b2_ref, q1w_ref, q1b_ref, q2w_ref,
               q2b_ref, i1w_ref, i1b_ref, i2w_ref, i2b_ref,
               q_ref, lsm_ref, i_ref):
    cnt = jnp.maximum(cnt_ref[:, :1], 1.0)
    pooled = jnp.dot(ps_ref[...] / cnt, w2_ref[...],
                     preferred_element_type=jnp.float32) + b2_ref[...]
    qh = jnp.maximum(
        jnp.dot(pooled, q1w_ref[...], preferred_element_type=jnp.float32)
        + q1b_ref[...], 0.0)
    q = jnp.dot(qh, q2w_ref[...],
                preferred_element_type=jnp.float32) + q2b_ref[...]
    ih = jnp.maximum(
        jnp.dot(pooled, i1w_ref[...], preferred_element_type=jnp.float32)
        + i1b_ref[...], 0.0)
    ii = jnp.dot(ih, i2w_ref[...],
                 preferred_element_type=jnp.float32) + i2b_ref[...]
    mx = jnp.max(ii, axis=1, keepdims=True)
    lse = jnp.log(jnp.sum(jnp.exp(ii - mx), axis=1, keepdims=True)) + mx
    q_ref[...] = q
    lsm_ref[...] = ii - lse
    i_ref[...] = ii


def _head(psum, cnt, W2, b2, q1W, q1b, q2W, q2b, i1W, i1b, i2W, i2b):
    return pl.pallas_call(
        _head_body,
        out_shape=[
            jax.ShapeDtypeStruct((G, A), jnp.float32),
            jax.ShapeDtypeStruct((G, A), jnp.float32),
            jax.ShapeDtypeStruct((G, A), jnp.float32),
        ],
    )(psum, cnt, W2, b2, q1W, q1b, q2W, q2b, i1W, i1b, i2W, i2b)


def kernel(x, edge_index, batch, W1, b1, W2, b2, q1W, q1b, q2W, q2b,
           i1W, i1b, i2W, i2b):
    f32 = jnp.float32
    src = edge_index[0]
    dst = edge_index[1]
    pad = EP - E
    padi = jnp.arange(pad, dtype=jnp.int32)
    pad_rows = N + (padi % (NP - N))
    src2d = jnp.concatenate([src, pad_rows]).reshape(EROWS, 128)
    dst2d = jnp.concatenate([dst, pad_rows]).reshape(EROWS, 128)

    degs = _deg_kernel(dst2d * 8, jnp.zeros((8 * NP, ), f32),
                       jnp.ones((128, ), f32))

    x_pad = jnp.pad(x, ((0, NP - N), (0, 0)))
    xs1, dis8 = _prep1(x_pad, degs.reshape(NC, NP, 8))
    acc1 = _prop1(xs1, src2d, dst2d)
    hs = _mid(acc1, xs1, dis8, W1, b1.reshape(1, H))

    batch_pad = jnp.pad(batch, (0, NP - N), constant_values=G)
    u = _upass(batch_pad * NP, dis8[:, 0], jnp.zeros((65 * NP, ), f32),
               src2d, dst2d)
    psum, cnt = _pool(hs, u.reshape(NC, G, NP), dis8,
                      batch_pad.reshape(1, NP))
    q, lsm, ii = _head(psum, cnt, W2, b2.reshape(1, H),
                       q1W, q1b.reshape(1, 64), q2W, q2b.reshape(1, A),
                       i1W, i1b.reshape(1, 64), i2W, i2b.reshape(1, A))
    return (q, lsm, ii)


# speedup vs baseline: 52.6155x; 1.0511x over previous
import functools

import jax
import jax.numpy as jnp
from jax import lax
from jax.experimental import pallas as pl
from jax.experimental.pallas import tpu as pltpu
from jax.experimental.pallas import tpu_sc as plsc

N = 10000
E = 320000
F = 128
H = 256
G = 64
A = 32

NC = 2
NS = 16
NP = 10240
RPT = NP // NS
T16 = 160
T32 = T16 // 2
EROWS = T16 * NS
EP = EROWS * 128

_mesh = plsc.VectorSubcoreMesh(core_axis_name="c", subcore_axis_name="s")


_DRT = 8 * NP // NS


@functools.partial(
    pl.kernel,
    out_type=jax.ShapeDtypeStruct((NC * 8 * NP, ), jnp.float32),
    mesh=_mesh,
    scratch_types=[
        pltpu.VMEM((T32, 128), jnp.int32),
        pltpu.VMEM((128, ), jnp.float32),
        pltpu.VMEM_SHARED((8 * NP, ), jnp.float32),
        pltpu.SemaphoreType.DMA,
    ],
)
def _deg_kernel(dst8_hbm, zeros_hbm, ones_hbm, out_hbm, idx_v, ones_v, acc,
                dsem):
    cid = lax.axis_index("c")
    sid = lax.axis_index("s")
    wid = cid * NS + sid
    pltpu.sync_copy(zeros_hbm.at[pl.ds(sid * _DRT, _DRT)],
                    acc.at[pl.ds(sid * _DRT, _DRT)])
    pltpu.sync_copy(ones_hbm, ones_v)
    pltpu.sync_copy(dst8_hbm.at[pl.ds(wid * T32, T32)], idx_v)
    plsc.subcore_barrier()

    def body(q, carry):
        base = q * 4
        descs = [
            pltpu.async_copy(ones_v, acc.at[idx_v.at[base + k]], dsem,
                             add=True)
            for k in range(4)
        ]
        for d in descs:
            d.wait()
        return carry

    lax.fori_loop(0, T32 // 4, body, 0)
    plsc.subcore_barrier()
    pltpu.sync_copy(acc.at[pl.ds(sid * _DRT, _DRT)],
                    out_hbm.at[pl.ds(cid * 8 * NP + sid * _DRT, _DRT)])


def _edge_pipeline(tab_hbm, src_hbm, dst_hbm, acc, src_v, dst_v, rows_a,
                   rows_b, gsa, gsb, ssa, ssb, src_base, dst_base, nblk,
                   blk_rows):
    npair = blk_rows // 2

    def outer(blk, carry):
        pltpu.sync_copy(
            src_hbm.at[pl.ds(src_base + blk * blk_rows, blk_rows)], src_v)
        pltpu.sync_copy(
            dst_hbm.at[pl.ds(dst_base + blk * blk_rows, blk_rows)], dst_v)
        pltpu.async_copy(tab_hbm.at[src_v.at[0]], rows_a, gsa)

        def pair(t, c2):
            j0 = 2 * t
            j1 = j0 + 1
            pltpu.async_copy(tab_hbm.at[src_v.at[j1]], rows_b, gsb)
            pltpu.make_async_copy(tab_hbm.at[src_v.at[j0]], rows_a,
                                  gsa).wait()
            sa = pltpu.async_copy(rows_a, acc.at[dst_v.at[j0]], ssa,
                                  add=True)
            sa.wait()

            @pl.when(t < npair - 1)
            def _():
                pltpu.async_copy(tab_hbm.at[src_v.at[j0 + 2]], rows_a, gsa)

            pltpu.make_async_copy(tab_hbm.at[src_v.at[j1]], rows_b,
                                  gsb).wait()
            sb = pltpu.async_copy(rows_b, acc.at[dst_v.at[j1]], ssb,
                                  add=True)
            sb.wait()
            return c2

        return lax.fori_loop(0, npair, pair, carry)

    lax.fori_loop(0, nblk, outer, 0)


@functools.partial(
    pl.kernel,
    out_type=jax.ShapeDtypeStruct((NC * NP, 128), jnp.float32),
    mesh=_mesh,
    scratch_types=[
        pltpu.VMEM((40, 128), jnp.int32),
        pltpu.VMEM((40, 128), jnp.int32),
        pltpu.VMEM((128, 128), jnp.float32),
        pltpu.VMEM((128, 128), jnp.float32),
        pltpu.VMEM_SHARED((NP, 128), jnp.float32),
        pltpu.SemaphoreType.DMA,
        pltpu.SemaphoreType.DMA,
        pltpu.SemaphoreType.DMA,
        pltpu.SemaphoreType.DMA,
    ],
)
def _prop1(xs_hbm, src_hbm, dst_hbm, out_hbm, src_v, dst_v,
           rows_a, rows_b, acc, gsa, gsb, ssa, ssb):
    cid = lax.axis_index("c")
    sid = lax.axis_index("s")
    wid = cid * NS + sid
    pltpu.sync_copy(xs_hbm.at[pl.ds(sid * RPT, RPT)],
                    acc.at[pl.ds(sid * RPT, RPT)])
    plsc.subcore_barrier()
    _edge_pipeline(xs_hbm, src_hbm, dst_hbm, acc, src_v, dst_v, rows_a,
                   rows_b, gsa, gsb, ssa, ssb,
                   src_base=wid * T32, dst_base=wid * T32, nblk=2,
                   blk_rows=40)
    plsc.subcore_barrier()
    pltpu.sync_copy(acc.at[pl.ds(sid * RPT, RPT)],
                    out_hbm.at[pl.ds(cid * NP + sid * RPT, RPT)])


_URT = 64 * NP // NS
_UZT = 65 * NP // NS
_NPS = NP // NS


@functools.partial(
    pl.kernel,
    out_type=jax.ShapeDtypeStruct((NC * 64 * NP, ), jnp.float32),
    mesh=_mesh,
    scratch_types=[
        pltpu.VMEM((T32, 128), jnp.int32),
        pltpu.VMEM((T32, 128), jnp.int32),
        pltpu.VMEM((128, ), jnp.int32),
        pltpu.VMEM((128, ), jnp.float32),
        pltpu.VMEM((128, ), jnp.int32),
        pltpu.VMEM((128, ), jnp.int32),
        pltpu.VMEM((128, ), jnp.float32),
        pltpu.VMEM((128, ), jnp.int32),
        pltpu.VMEM_SHARED((65 * NP, ), jnp.float32),
        pltpu.VMEM_SHARED((NP, ), jnp.int32),
        pltpu.VMEM_SHARED((NP, ), jnp.float32),
        pltpu.SemaphoreType.DMA,
        pltpu.SemaphoreType.DMA,
        pltpu.SemaphoreType.DMA,
        pltpu.SemaphoreType.DMA,
        pltpu.SemaphoreType.DMA,
        pltpu.SemaphoreType.DMA,
    ],
)
def _upass(key_hbm, dis_hbm, zeros_hbm, src_hbm, dst_hbm, out_hbm,
           dst_v, src_v, krow_a, vrow_a, flat_a, krow_b, vrow_b, flat_b,
           acc, key_sp, dis_sp, gka, gva, gkb, gvb, sca, scb):
    cid = lax.axis_index("c")
    sid = lax.axis_index("s")
    wid = cid * NS + sid
    pltpu.sync_copy(zeros_hbm.at[pl.ds(sid * _UZT, _UZT)],
                    acc.at[pl.ds(sid * _UZT, _UZT)])
    pltpu.sync_copy(key_hbm.at[pl.ds(sid * _NPS, _NPS)],
                    key_sp.at[pl.ds(sid * _NPS, _NPS)])
    pltpu.sync_copy(dis_hbm.at[pl.ds(sid * _NPS, _NPS)],
                    dis_sp.at[pl.ds(sid * _NPS, _NPS)])
    pltpu.sync_copy(dst_hbm.at[pl.ds(wid * T32, T32)], dst_v)
    pltpu.sync_copy(src_hbm.at[pl.ds(wid * T32, T32)], src_v)
    plsc.subcore_barrier()

    def _half(j, krow, vrow, flat_v, gk, gv, sc):
        pltpu.make_async_copy(key_sp.at[dst_v.at[j]], krow, gk).wait()
        pltpu.make_async_copy(dis_sp.at[dst_v.at[j]], vrow, gv).wait()
        for k in range(8):
            sl = pl.ds(k * 16, 16)
            flat_v[sl] = krow[sl] + src_v[j, sl]
        pltpu.async_copy(vrow, acc.at[flat_v], sc, add=True).wait()

    pltpu.async_copy(key_sp.at[dst_v.at[0]], krow_a, gka)
    pltpu.async_copy(dis_sp.at[dst_v.at[0]], vrow_a, gva)

    def pair(t, carry):
        j0 = 2 * t
        j1 = j0 + 1
        pltpu.async_copy(key_sp.at[dst_v.at[j1]], krow_b, gkb)
        pltpu.async_copy(dis_sp.at[dst_v.at[j1]], vrow_b, gvb)
        _half(j0, krow_a, vrow_a, flat_a, gka, gva, sca)

        @pl.when(t < T32 // 2 - 1)
        def _():
            pltpu.async_copy(key_sp.at[dst_v.at[j0 + 2]], krow_a, gka)
            pltpu.async_copy(dis_sp.at[dst_v.at[j0 + 2]], vrow_a, gva)

        _half(j1, krow_b, vrow_b, flat_b, gkb, gvb, scb)
        return carry

    lax.fori_loop(0, T32 // 2, pair, 0)
    plsc.subcore_barrier()
    pltpu.sync_copy(acc.at[pl.ds(sid * _URT, _URT)],
                    out_hbm.at[pl.ds(cid * 64 * NP + sid * _URT, _URT)])


_NB = 8
_BR = NP // _NB


def _prep1_body(x_ref, dega_ref, degb_ref, xs_ref, dis_ref):
    deg = dega_ref[0][:, :1] + degb_ref[0][:, :1] + 1.0
    dis = lax.rsqrt(deg)
    xs_ref[...] = x_ref[...] * dis
    dis_ref[...] = jnp.broadcast_to(dis, (_BR, 8))


def _prep1(x_pad, degs3):
    return pl.pallas_call(
        _prep1_body,
        grid=(_NB,),
        in_specs=[
            pl.BlockSpec((_BR, F), lambda i: (i, 0)),
            pl.BlockSpec((1, _BR, 8), lambda i: (0, i, 0)),
            pl.BlockSpec((1, _BR, 8), lambda i: (1, i, 0)),
        ],
        out_specs=[
            pl.BlockSpec((_BR, F), lambda i: (i, 0)),
            pl.BlockSpec((_BR, 8), lambda i: (i, 0)),
        ],
        out_shape=[
            jax.ShapeDtypeStruct((NP, F), jnp.float32),
            jax.ShapeDtypeStruct((NP, 8), jnp.float32),
        ],
    )(x_pad, degs3, degs3)


def _final_body(a_ref, b_ref, xs_ref, dis_ref, w1_ref, b1_ref,
                u0_ref, u1_ref, bt_ref, w2_ref, b2_ref, q1w_ref, q1b_ref,
                q2w_ref, q2b_ref, i1w_ref, i1b_ref, i2w_ref, i2b_ref,
                psum_ref, cnt_ref, q_ref, lsm_ref, i_ref):
    i = pl.program_id(0)

    @pl.when(i == 0)
    def _():
        psum_ref[...] = jnp.zeros_like(psum_ref)
        cnt_ref[...] = jnp.zeros_like(cnt_ref)

    dis = dis_ref[:, :1]
    y = (a_ref[...] + b_ref[...] - xs_ref[...]) * dis
    h = jnp.dot(y, w1_ref[...], preferred_element_type=jnp.float32)
    h = jnp.maximum(h + b1_ref[...], 0.0) * dis
    bt = bt_ref[...]
    gi = lax.broadcasted_iota(jnp.int32, (G, _BR), 0)
    m = (bt == gi).astype(jnp.float32)
    u = u0_ref[0] + u1_ref[0]
    psum_ref[...] += (
        jnp.dot(u, h, preferred_element_type=jnp.float32)
        + jnp.dot(m, h * dis, preferred_element_type=jnp.float32))
    cnt_ref[...] += jnp.broadcast_to(jnp.sum(m, axis=1, keepdims=True),
                                     (G, 128))

    @pl.when(i == _NB - 1)
    def _():
        cntc = jnp.maximum(cnt_ref[:, :1], 1.0)
        pooled = jnp.dot(psum_ref[...] / cntc, w2_ref[...],
                         preferred_element_type=jnp.float32) + b2_ref[...]
        qh = jnp.maximum(
            jnp.dot(pooled, q1w_ref[...], preferred_element_type=jnp.float32)
            + q1b_ref[...], 0.0)
        q_ref[...] = jnp.dot(qh, q2w_ref[...],
                             preferred_element_type=jnp.float32) + q2b_ref[...]
        ih = jnp.maximum(
            jnp.dot(pooled, i1w_ref[...], preferred_element_type=jnp.float32)
            + i1b_ref[...], 0.0)
        ii = jnp.dot(ih, i2w_ref[...],
                     preferred_element_type=jnp.float32) + i2b_ref[...]
        mx = jnp.max(ii, axis=1, keepdims=True)
        lse = jnp.log(jnp.sum(jnp.exp(ii - mx), axis=1, keepdims=True)) + mx
        lsm_ref[...] = ii - lse
        i_ref[...] = ii


def _final(acc1, xs1, dis8, W1, b1, u3, batch2d, W2, b2,
           q1W, q1b, q2W, q2b, i1W, i1b, i2W, i2b):
    full = lambda shp: pl.BlockSpec(shp, lambda i: tuple(0 for _ in shp))
    return pl.pallas_call(
        _final_body,
        grid=(_NB,),
        in_specs=[
            pl.BlockSpec((_BR, 128), lambda i: (i, 0)),
            pl.BlockSpec((_BR, 128), lambda i: (i + _NB, 0)),
            pl.BlockSpec((_BR, F), lambda i: (i, 0)),
            pl.BlockSpec((_BR, 8), lambda i: (i, 0)),
            full((F, H)),
            full((1, H)),
            pl.BlockSpec((1, G, _BR), lambda i: (0, 0, i)),
            pl.BlockSpec((1, G, _BR), lambda i: (1, 0, i)),
            pl.BlockSpec((1, _BR), lambda i: (0, i)),
            full((H, H)),
            full((1, H)),
            full((H, 64)),
            full((1, 64)),
            full((64, A)),
            full((1, A)),
            full((H, 64)),
            full((1, 64)),
            full((64, A)),
            full((1, A)),
        ],
        out_specs=[
            full((G, H)),
            full((G, 128)),
            full((G, A)),
            full((G, A)),
            full((G, A)),
        ],
        out_shape=[
            jax.ShapeDtypeStruct((G, H), jnp.float32),
            jax.ShapeDtypeStruct((G, 128), jnp.float32),
            jax.ShapeDtypeStruct((G, A), jnp.float32),
            jax.ShapeDtypeStruct((G, A), jnp.float32),
            jax.ShapeDtypeStruct((G, A), jnp.float32),
        ],
    )(acc1, acc1, xs1, dis8, W1, b1, u3, u3, batch2d, W2, b2,
      q1W, q1b, q2W, q2b, i1W, i1b, i2W, i2b)


def kernel(x, edge_index, batch, W1, b1, W2, b2, q1W, q1b, q2W, q2b,
           i1W, i1b, i2W, i2b):
    f32 = jnp.float32
    src = edge_index[0]
    dst = edge_index[1]
    pad = EP - E
    padi = jnp.arange(pad, dtype=jnp.int32)
    pad_rows = N + (padi % (NP - N))
    src2d = jnp.concatenate([src, pad_rows]).reshape(EROWS, 128)
    dst2d = jnp.concatenate([dst, pad_rows]).reshape(EROWS, 128)

    degs = _deg_kernel(dst2d * 8, jnp.zeros((8 * NP, ), f32),
                       jnp.ones((128, ), f32))

    x_pad = jnp.pad(x, ((0, NP - N), (0, 0)))
    xs1, dis8 = _prep1(x_pad, degs.reshape(NC, NP, 8))
    acc1 = _prop1(xs1, src2d, dst2d)

    batch_pad = jnp.pad(batch, (0, NP - N), constant_values=G)
    u = _upass(batch_pad * NP, dis8[:, 0], jnp.zeros((65 * NP, ), f32),
               src2d, dst2d)
    _, _, q, lsm, ii = _final(
        acc1, xs1, dis8, W1, b1.reshape(1, H), u.reshape(NC, G, NP),
        batch_pad.reshape(1, NP), W2, b2.reshape(1, H),
        q1W, q1b.reshape(1, 64), q2W, q2b.reshape(1, A),
        i1W, i1b.reshape(1, 64), i2W, i2b.reshape(1, A))
    return (q, lsm, ii)
